# Initial kernel scaffold; baseline (speedup 1.0000x reference)
import jax
import jax.numpy as jnp
from jax.experimental import pallas as pl


def kernel(xyz):
    raise NotImplementedError("write your pallas kernel here")


# speedup vs baseline: 2.7432x; 2.7432x over previous
import functools

import jax
import jax.numpy as jnp
from jax.experimental import pallas as pl
from jax.experimental.pallas import tpu as pltpu

NG = 512
GS = 32
GB = 8
FR = 8


def _fps_body(x_ref, y_ref, z_ref, cx_ref, cy_ref, cz_ref, dists_ref):
    _, R, C = x_ref.shape
    n_total = R * C
    x = x_ref[0]
    y = y_ref[0]
    z = z_ref[0]
    iota = (jax.lax.broadcasted_iota(jnp.int32, (R, C), 0) * C
            + jax.lax.broadcasted_iota(jnp.int32, (R, C), 1))
    inf = jnp.float32(jnp.inf)
    dists_ref[...] = jnp.full((R, C), inf, dtype=jnp.float32)

    def red2(a, op):
        return op(op(a, axis=1, keepdims=True), axis=0, keepdims=True)

    giota = jax.lax.broadcasted_iota(jnp.int32, (1, NG), 1)

    def body(i, carry):
        cur, acx, acy, acz = carry
        mask = iota == cur
        cxv = red2(jnp.where(mask, x, 0.0), jnp.sum)
        cyv = red2(jnp.where(mask, y, 0.0), jnp.sum)
        czv = red2(jnp.where(mask, z, 0.0), jnp.sum)
        sel = giota == i
        acx = jnp.where(sel, cxv, acx)
        acy = jnp.where(sel, cyv, acy)
        acz = jnp.where(sel, czv, acz)
        d = ((x - cxv) ** 2 + (y - cyv) ** 2) + (z - czv) ** 2
        dn = jnp.minimum(dists_ref[...], d)
        dists_ref[...] = dn
        m = red2(dn, jnp.max)
        cand = jnp.where(dn == m, iota, jnp.int32(n_total))
        return red2(cand, jnp.min), acx, acy, acz

    zc = jnp.zeros((1, NG), jnp.float32)
    _, acx, acy, acz = jax.lax.fori_loop(
        0, NG, body, (jnp.zeros((1, 1), jnp.int32), zc, zc, zc))
    cx_ref[0] = acx
    cy_ref[0] = acy
    cz_ref[0] = acz


def _knn_body(x_ref, y_ref, z_ref, cx_ref, cy_ref, cz_ref,
              knn_ref, nx_ref, ny_ref, nz_ref, dist_ref):
    _, _, n = x_ref.shape
    b = pl.program_id(0)
    x = x_ref[0]
    y = y_ref[0]
    z = z_ref[0]
    cx = cx_ref[0, 0]
    cy = cy_ref[0, 0]
    cz = cz_ref[0, 0]
    b2 = (x * x + y * y) + z * z
    a2 = (cx * cx + cy * cy) + cz * cz

    def bf(v):
        return v.astype(jnp.bfloat16).astype(jnp.float32)

    ab = (bf(cx) * bf(x) + bf(cy) * bf(y)) + bf(cz) * bf(z)
    d2 = jnp.maximum(a2 + b2 - 2.0 * ab, 0.0)
    dist_ref[...] = jnp.sqrt(d2)
    iota = jax.lax.broadcasted_iota(jnp.int32, (GB, n), 1)
    kiota = jax.lax.broadcasted_iota(jnp.int32, (GB, GS), 1)
    inf = jnp.float32(jnp.inf)

    def body(k, carry):
        aknn, anx, any_, anz = carry
        d = dist_ref[...]
        m = jnp.min(d, axis=1, keepdims=True)
        cand = jnp.where(d == m, iota, jnp.int32(n))
        idx = jnp.min(cand, axis=1, keepdims=True)
        mask2 = iota == idx
        sel = kiota == k
        aknn = jnp.where(sel, idx + b * n, aknn)
        anx = jnp.where(sel, jnp.sum(jnp.where(mask2, x, 0.0), axis=1,
                                     keepdims=True), anx)
        any_ = jnp.where(sel, jnp.sum(jnp.where(mask2, y, 0.0), axis=1,
                                      keepdims=True), any_)
        anz = jnp.where(sel, jnp.sum(jnp.where(mask2, z, 0.0), axis=1,
                                     keepdims=True), anz)
        dist_ref[...] = jnp.where(mask2, inf, d)
        return aknn, anx, any_, anz

    zi = jnp.zeros((GB, GS), jnp.int32)
    zf = jnp.zeros((GB, GS), jnp.float32)
    aknn, anx, any_, anz = jax.lax.fori_loop(0, GS, body, (zi, zf, zf, zf))
    knn_ref[0] = aknn
    nx_ref[0] = anx
    ny_ref[0] = any_
    nz_ref[0] = anz


@jax.jit
def kernel(xyz):
    B, N, _ = xyz.shape
    x = xyz[:, :, 0]
    y = xyz[:, :, 1]
    z = xyz[:, :, 2]
    xr = x.reshape(B, FR, N // FR)
    yr = y.reshape(B, FR, N // FR)
    zr = z.reshape(B, FR, N // FR)

    cplanes = pl.pallas_call(
        _fps_body,
        grid=(B,),
        in_specs=[pl.BlockSpec((1, FR, N // FR), lambda b: (b, 0, 0))] * 3,
        out_specs=[pl.BlockSpec((1, 1, NG), lambda b: (b, 0, 0))] * 3,
        out_shape=[jax.ShapeDtypeStruct((B, 1, NG), jnp.float32)] * 3,
        scratch_shapes=[pltpu.VMEM((FR, N // FR), jnp.float32)],
    )(xr, yr, zr)
    cx, cy, cz = (c.reshape(B, NG) for c in cplanes)
    cxr = cx.reshape(B, NG // GB, GB, 1)
    cyr = cy.reshape(B, NG // GB, GB, 1)
    czr = cz.reshape(B, NG // GB, GB, 1)

    knn, nx, ny, nz = pl.pallas_call(
        _knn_body,
        grid=(B, NG // GB),
        in_specs=(
            [pl.BlockSpec((1, 1, N), lambda b, g: (b, 0, 0))] * 3
            + [pl.BlockSpec((1, 1, GB, 1), lambda b, g: (b, g, 0, 0))] * 3
        ),
        out_specs=[pl.BlockSpec((1, GB, GS), lambda b, g: (b, g, 0))] * 4,
        out_shape=(
            [jax.ShapeDtypeStruct((B, NG, GS), jnp.int32)]
            + [jax.ShapeDtypeStruct((B, NG, GS), jnp.float32)] * 3
        ),
        scratch_shapes=[pltpu.VMEM((GB, N), jnp.float32)],
    )(x.reshape(B, 1, N), y.reshape(B, 1, N), z.reshape(B, 1, N),
      cxr, cyr, czr)

    knn_idx_flat = knn.reshape(-1)
    nbr_xyz = jnp.stack([nx, ny, nz], axis=-1)
    return knn_idx_flat, nbr_xyz


# speedup vs baseline: 3.9662x; 1.4458x over previous
import functools

import jax
import jax.numpy as jnp
from jax import lax
from jax.experimental import pallas as pl
from jax.experimental.pallas import tpu as pltpu
from jax.experimental.pallas import tpu_sc as plsc

NG = 512
GS = 32
GB = 8
FR = 8


def _fps_body(x_ref, y_ref, z_ref, cx_ref, cy_ref, cz_ref, dists_ref):
    _, R, C = x_ref.shape
    n_total = R * C
    x = x_ref[0]
    y = y_ref[0]
    z = z_ref[0]
    iota = (jax.lax.broadcasted_iota(jnp.int32, (R, C), 0) * C
            + jax.lax.broadcasted_iota(jnp.int32, (R, C), 1))
    inf = jnp.float32(jnp.inf)
    dists_ref[...] = jnp.full((R, C), inf, dtype=jnp.float32)

    def red2(a, op):
        return op(op(a, axis=1, keepdims=True), axis=0, keepdims=True)

    giota = jax.lax.broadcasted_iota(jnp.int32, (1, NG), 1)

    def body(i, carry):
        cur, acx, acy, acz = carry
        mask = iota == cur
        cxv = red2(jnp.where(mask, x, 0.0), jnp.sum)
        cyv = red2(jnp.where(mask, y, 0.0), jnp.sum)
        czv = red2(jnp.where(mask, z, 0.0), jnp.sum)
        sel = giota == i
        acx = jnp.where(sel, cxv, acx)
        acy = jnp.where(sel, cyv, acy)
        acz = jnp.where(sel, czv, acz)
        d = ((x - cxv) ** 2 + (y - cyv) ** 2) + (z - czv) ** 2
        dn = jnp.minimum(dists_ref[...], d)
        dists_ref[...] = dn
        m = red2(dn, jnp.max)
        cand = jnp.where(dn == m, iota, jnp.int32(n_total))
        return red2(cand, jnp.min), acx, acy, acz

    zc = jnp.zeros((1, NG), jnp.float32)
    _, acx, acy, acz = jax.lax.fori_loop(
        0, NG, body, (jnp.zeros((1, 1), jnp.int32), zc, zc, zc))
    cx_ref[0] = acx
    cy_ref[0] = acy
    cz_ref[0] = acz


def _knn_body(x_ref, y_ref, z_ref, cx_ref, cy_ref, cz_ref,
              knn_ref, dist_ref):
    _, _, n = x_ref.shape
    b = pl.program_id(0)
    x = x_ref[0]
    y = y_ref[0]
    z = z_ref[0]
    cx = cx_ref[0, 0]
    cy = cy_ref[0, 0]
    cz = cz_ref[0, 0]
    b2 = (x * x + y * y) + z * z
    a2 = (cx * cx + cy * cy) + cz * cz

    def bf(v):
        return v.astype(jnp.bfloat16).astype(jnp.float32)

    ab = (bf(cx) * bf(x) + bf(cy) * bf(y)) + bf(cz) * bf(z)
    d2 = jnp.maximum(a2 + b2 - 2.0 * ab, 0.0)
    dist_ref[...] = jnp.sqrt(d2)
    iota = jax.lax.broadcasted_iota(jnp.int32, (GB, n), 1)
    kiota = jax.lax.broadcasted_iota(jnp.int32, (GB, GS), 1)
    inf = jnp.float32(jnp.inf)

    def body(k, aknn):
        d = dist_ref[...]
        m = jnp.min(d, axis=1, keepdims=True)
        cand = jnp.where(d == m, iota, jnp.int32(n))
        idx = jnp.min(cand, axis=1, keepdims=True)
        mask2 = iota == idx
        aknn = jnp.where(kiota == k, idx + b * n, aknn)
        dist_ref[...] = jnp.where(mask2, inf, d)
        return aknn

    zi = jnp.zeros((GB, GS), jnp.int32)
    knn_ref[0] = jax.lax.fori_loop(0, GS, body, zi)


def _make_sc_gather(total, pad):
    info = plsc.get_sparse_core_info()
    nw = info.num_cores * info.num_subcores
    rpw = total // nw

    @functools.partial(
        pl.kernel,
        mesh=plsc.VectorSubcoreMesh(core_axis_name="c", subcore_axis_name="s"),
        compiler_params=pltpu.CompilerParams(use_tc_tiling_on_sc=False),
        out_type=jax.ShapeDtypeStruct((total, pad), jnp.float32),
        scratch_types=[
            pltpu.VMEM((rpw,), jnp.int32),
            pltpu.VMEM((rpw, pad), jnp.float32),
            pltpu.SemaphoreType.DMA,
        ],
    )
    def gather_k(idx_hbm, tab_hbm, out_hbm, idx_v, rows_v, sem):
        wid = lax.axis_index("s") * info.num_cores + lax.axis_index("c")
        base = wid * rpw
        pltpu.sync_copy(idx_hbm.at[pl.ds(base, rpw)], idx_v)
        pltpu.async_copy(tab_hbm.at[idx_v], rows_v, sem).wait()
        pltpu.sync_copy(rows_v, out_hbm.at[pl.ds(base, rpw)])

    return gather_k


@jax.jit
def kernel(xyz):
    B, N, _ = xyz.shape
    x = xyz[:, :, 0]
    y = xyz[:, :, 1]
    z = xyz[:, :, 2]
    xr = x.reshape(B, FR, N // FR)
    yr = y.reshape(B, FR, N // FR)
    zr = z.reshape(B, FR, N // FR)

    cplanes = pl.pallas_call(
        _fps_body,
        grid=(B,),
        in_specs=[pl.BlockSpec((1, FR, N // FR), lambda b: (b, 0, 0))] * 3,
        out_specs=[pl.BlockSpec((1, 1, NG), lambda b: (b, 0, 0))] * 3,
        out_shape=[jax.ShapeDtypeStruct((B, 1, NG), jnp.float32)] * 3,
        scratch_shapes=[pltpu.VMEM((FR, N // FR), jnp.float32)],
    )(xr, yr, zr)
    cx, cy, cz = (c.reshape(B, NG) for c in cplanes)
    cxr = cx.reshape(B, NG // GB, GB, 1)
    cyr = cy.reshape(B, NG // GB, GB, 1)
    czr = cz.reshape(B, NG // GB, GB, 1)

    knn = pl.pallas_call(
        _knn_body,
        grid=(B, NG // GB),
        in_specs=(
            [pl.BlockSpec((1, 1, N), lambda b, g: (b, 0, 0))] * 3
            + [pl.BlockSpec((1, 1, GB, 1), lambda b, g: (b, g, 0, 0))] * 3
        ),
        out_specs=pl.BlockSpec((1, GB, GS), lambda b, g: (b, g, 0)),
        out_shape=jax.ShapeDtypeStruct((B, NG, GS), jnp.int32),
        scratch_shapes=[pltpu.VMEM((GB, N), jnp.float32)],
    )(x.reshape(B, 1, N), y.reshape(B, 1, N), z.reshape(B, 1, N),
      cxr, cyr, czr)

    knn_idx_flat = knn.reshape(-1)
    total = B * NG * GS
    pad = 8
    table = jnp.concatenate(
        [xyz.reshape(B * N, 3),
         jnp.zeros((B * N, pad - 3), jnp.float32)], axis=1)
    rows = _make_sc_gather(total, pad)(knn_idx_flat, table)
    nbr_xyz = rows[:, :3].reshape(B, NG, GS, 3)
    return knn_idx_flat, nbr_xyz


# speedup vs baseline: 4.1893x; 1.0563x over previous
import functools

import jax
import jax.numpy as jnp
from jax import lax
from jax.experimental import pallas as pl
from jax.experimental.pallas import tpu as pltpu
from jax.experimental.pallas import tpu_sc as plsc

NG = 512
GS = 32
GB = 8
FR = 8


def _fps_body(x_ref, y_ref, z_ref, cx_ref, cy_ref, cz_ref, dists_ref):
    _, R, C = x_ref.shape
    n_total = R * C
    x = x_ref[0]
    y = y_ref[0]
    z = z_ref[0]
    iota = (jax.lax.broadcasted_iota(jnp.int32, (R, C), 0) * C
            + jax.lax.broadcasted_iota(jnp.int32, (R, C), 1))
    inf = jnp.float32(jnp.inf)
    dists_ref[...] = jnp.full((R, C), inf, dtype=jnp.float32)

    def red2(a, op):
        return op(op(a, axis=1, keepdims=True), axis=0, keepdims=True)

    giota = jax.lax.broadcasted_iota(jnp.int32, (1, NG), 1)

    def body(i, carry):
        cur, acx, acy, acz = carry
        mask = iota == cur
        cxv = red2(jnp.where(mask, x, 0.0), jnp.sum)
        cyv = red2(jnp.where(mask, y, 0.0), jnp.sum)
        czv = red2(jnp.where(mask, z, 0.0), jnp.sum)
        sel = giota == i
        acx = jnp.where(sel, cxv, acx)
        acy = jnp.where(sel, cyv, acy)
        acz = jnp.where(sel, czv, acz)
        d = ((x - cxv) ** 2 + (y - cyv) ** 2) + (z - czv) ** 2
        dn = jnp.minimum(dists_ref[...], d)
        dists_ref[...] = dn
        m = red2(dn, jnp.max)
        cand = jnp.where(dn == m, iota, jnp.int32(n_total))
        return red2(cand, jnp.min), acx, acy, acz

    zc = jnp.zeros((1, NG), jnp.float32)
    _, acx, acy, acz = jax.lax.fori_loop(
        0, NG, body, (jnp.zeros((1, 1), jnp.int32), zc, zc, zc))
    cx_ref[0] = acx
    cy_ref[0] = acy
    cz_ref[0] = acz


NJ = 128
NL = 128


def _knn_body(x_ref, y_ref, z_ref, cx_ref, cy_ref, cz_ref,
              knn_ref, dist_ref):
    n_total = NJ * NL
    b = pl.program_id(0)
    x = x_ref[...]
    y = y_ref[...]
    z = z_ref[...]
    cx = cx_ref[0, 0].reshape(GB, 1, 1)
    cy = cy_ref[0, 0].reshape(GB, 1, 1)
    cz = cz_ref[0, 0].reshape(GB, 1, 1)
    b2 = (x * x + y * y) + z * z
    a2 = (cx * cx + cy * cy) + cz * cz

    def bf(v):
        return v.astype(jnp.bfloat16).astype(jnp.float32)

    ab = (bf(cx) * bf(x) + bf(cy) * bf(y)) + bf(cz) * bf(z)
    d2 = jnp.maximum(a2 + b2 - 2.0 * ab, 0.0)
    dist_ref[...] = jnp.sqrt(d2)

    n3 = (jax.lax.broadcasted_iota(jnp.int32, (GB, NJ, NL), 1) * NL
          + jax.lax.broadcasted_iota(jnp.int32, (GB, NJ, NL), 2))
    laneio = jax.lax.broadcasted_iota(jnp.int32, (GB, NL), 1)
    kiota = jax.lax.broadcasted_iota(jnp.int32, (GB, GS), 1)
    inf = jnp.float32(jnp.inf)
    nbig = jnp.int32(n_total)

    def build(mk, nk):
        d = dist_ref[...]
        mkb = mk.reshape(GB, 1, 1)
        nkb = nk.reshape(GB, 1, 1)
        dx = jnp.where((d < mkb) | ((d == mkb) & (n3 <= nkb)), inf, d)
        m1 = jnp.min(dx, axis=1)
        a1 = jnp.min(jnp.where(dx == m1[:, None, :], n3, nbig), axis=1)
        dx = jnp.where(n3 == a1[:, None, :], inf, dx)
        m2 = jnp.min(dx, axis=1)
        a2_ = jnp.min(jnp.where(dx == m2[:, None, :], n3, nbig), axis=1)
        dx = jnp.where(n3 == a2_[:, None, :], inf, dx)
        m3 = jnp.min(dx, axis=1)
        a3 = jnp.min(jnp.where(dx == m3[:, None, :], n3, nbig), axis=1)
        c = ((m1 != inf).astype(jnp.int32) + (m2 != inf).astype(jnp.int32)
             + (m3 != inf).astype(jnp.int32))
        return m1, a1, m2, a2_, m3, a3, c

    def body(k, carry):
        aknn, m1, a1, m2, a2_, m3, a3, c = carry
        m = jnp.min(m1, axis=1, keepdims=True)
        idx = jnp.min(jnp.where(m1 == m, a1, nbig), axis=1,
                      keepdims=True)
        aknn = jnp.where(kiota == k, idx + b * n_total, aknn)
        sel = laneio == jnp.bitwise_and(idx, jnp.int32(NL - 1))
        m1n = jnp.where(sel, m2, m1)
        a1n = jnp.where(sel, a2_, a1)
        m2n = jnp.where(sel, m3, m2)
        a2n = jnp.where(sel, a3, a2_)
        m3n = jnp.where(sel, inf, m3)
        a3n = jnp.where(sel, nbig, a3)
        cn = c - sel.astype(jnp.int32)
        need = jnp.min(cn) == 0
        caches = jax.lax.cond(
            need,
            lambda: build(m, idx),
            lambda: (m1n, a1n, m2n, a2n, m3n, a3n, cn))
        return (aknn,) + caches

    init = build(jnp.full((GB, 1), -jnp.inf, jnp.float32),
                 jnp.full((GB, 1), -1, jnp.int32))
    zi = jnp.zeros((GB, GS), jnp.int32)
    out = jax.lax.fori_loop(0, GS, body, (zi,) + init)
    knn_ref[0] = out[0]


def _make_sc_gather(total, pad):
    info = plsc.get_sparse_core_info()
    nw = info.num_cores * info.num_subcores
    rpw = total // nw

    @functools.partial(
        pl.kernel,
        mesh=plsc.VectorSubcoreMesh(core_axis_name="c", subcore_axis_name="s"),
        compiler_params=pltpu.CompilerParams(use_tc_tiling_on_sc=False),
        out_type=jax.ShapeDtypeStruct((total, pad), jnp.float32),
        scratch_types=[
            pltpu.VMEM((rpw,), jnp.int32),
            pltpu.VMEM((rpw, pad), jnp.float32),
            pltpu.SemaphoreType.DMA,
        ],
    )
    def gather_k(idx_hbm, tab_hbm, out_hbm, idx_v, rows_v, sem):
        wid = lax.axis_index("s") * info.num_cores + lax.axis_index("c")
        base = wid * rpw
        pltpu.sync_copy(idx_hbm.at[pl.ds(base, rpw)], idx_v)
        pltpu.async_copy(tab_hbm.at[idx_v], rows_v, sem).wait()
        pltpu.sync_copy(rows_v, out_hbm.at[pl.ds(base, rpw)])

    return gather_k


@jax.jit
def kernel(xyz):
    B, N, _ = xyz.shape
    x = xyz[:, :, 0]
    y = xyz[:, :, 1]
    z = xyz[:, :, 2]
    xr = x.reshape(B, FR, N // FR)
    yr = y.reshape(B, FR, N // FR)
    zr = z.reshape(B, FR, N // FR)

    cplanes = pl.pallas_call(
        _fps_body,
        grid=(B,),
        in_specs=[pl.BlockSpec((1, FR, N // FR), lambda b: (b, 0, 0))] * 3,
        out_specs=[pl.BlockSpec((1, 1, NG), lambda b: (b, 0, 0))] * 3,
        out_shape=[jax.ShapeDtypeStruct((B, 1, NG), jnp.float32)] * 3,
        scratch_shapes=[pltpu.VMEM((FR, N // FR), jnp.float32)],
    )(xr, yr, zr)
    cx, cy, cz = (c.reshape(B, NG) for c in cplanes)
    cxr = cx.reshape(B, NG // GB, GB, 1)
    cyr = cy.reshape(B, NG // GB, GB, 1)
    czr = cz.reshape(B, NG // GB, GB, 1)

    knn = pl.pallas_call(
        _knn_body,
        grid=(B, NG // GB),
        in_specs=(
            [pl.BlockSpec((1, NJ, NL), lambda b, g: (b, 0, 0))] * 3
            + [pl.BlockSpec((1, 1, GB, 1), lambda b, g: (b, g, 0, 0))] * 3
        ),
        out_specs=pl.BlockSpec((1, GB, GS), lambda b, g: (b, g, 0)),
        out_shape=jax.ShapeDtypeStruct((B, NG, GS), jnp.int32),
        scratch_shapes=[pltpu.VMEM((GB, NJ, NL), jnp.float32)],
    )(x.reshape(B, NJ, NL), y.reshape(B, NJ, NL), z.reshape(B, NJ, NL),
      cxr, cyr, czr)

    knn_idx_flat = knn.reshape(-1)
    total = B * NG * GS
    pad = 8
    table = jnp.concatenate(
        [xyz.reshape(B * N, 3),
         jnp.zeros((B * N, pad - 3), jnp.float32)], axis=1)
    rows = _make_sc_gather(total, pad)(knn_idx_flat, table)
    nbr_xyz = rows[:, :3].reshape(B, NG, GS, 3)
    return knn_idx_flat, nbr_xyz


# speedup vs baseline: 4.2204x; 1.0074x over previous
import functools

import jax
import jax.numpy as jnp
from jax import lax
from jax.experimental import pallas as pl
from jax.experimental.pallas import tpu as pltpu
from jax.experimental.pallas import tpu_sc as plsc

NG = 512
GS = 32
GB = 8
FR = 8


def _fps_body(x_ref, y_ref, z_ref, cx_ref, cy_ref, cz_ref, dists_ref):
    _, R, C = x_ref.shape
    n_total = R * C
    x = x_ref[0]
    y = y_ref[0]
    z = z_ref[0]
    iota = (jax.lax.broadcasted_iota(jnp.int32, (R, C), 0) * C
            + jax.lax.broadcasted_iota(jnp.int32, (R, C), 1))
    inf = jnp.float32(jnp.inf)
    dists_ref[...] = jnp.full((R, C), inf, dtype=jnp.float32)

    def red2(a, op):
        return op(op(a, axis=1, keepdims=True), axis=0, keepdims=True)

    giota = jax.lax.broadcasted_iota(jnp.int32, (1, NG), 1)

    def body(i, carry):
        cur, acx, acy, acz = carry
        mask = iota == cur
        cxv = red2(jnp.where(mask, x, 0.0), jnp.sum)
        cyv = red2(jnp.where(mask, y, 0.0), jnp.sum)
        czv = red2(jnp.where(mask, z, 0.0), jnp.sum)
        sel = giota == i
        acx = jnp.where(sel, cxv, acx)
        acy = jnp.where(sel, cyv, acy)
        acz = jnp.where(sel, czv, acz)
        d = ((x - cxv) ** 2 + (y - cyv) ** 2) + (z - czv) ** 2
        dn = jnp.minimum(dists_ref[...], d)
        dists_ref[...] = dn
        m = red2(dn, jnp.max)
        cand = jnp.where(dn == m, iota, jnp.int32(n_total))
        return red2(cand, jnp.min), acx, acy, acz

    zc = jnp.zeros((1, NG), jnp.float32)
    _, acx, acy, acz = jax.lax.fori_loop(
        0, NG, body, (jnp.zeros((1, 1), jnp.int32), zc, zc, zc))
    cx_ref[0] = acx
    cy_ref[0] = acy
    cz_ref[0] = acz


NJ = 128
NL = 128


def _knn_body(x_ref, y_ref, z_ref, cx_ref, cy_ref, cz_ref,
              knn_ref, dist_ref, m1_ref, a1_ref, m2_ref, a2_ref,
              m3_ref, a3_ref, c_ref):
    n_total = NJ * NL
    b = pl.program_id(0)
    x = x_ref[...]
    y = y_ref[...]
    z = z_ref[...]
    cx = cx_ref[0, 0].reshape(GB, 1, 1)
    cy = cy_ref[0, 0].reshape(GB, 1, 1)
    cz = cz_ref[0, 0].reshape(GB, 1, 1)
    b2 = (x * x + y * y) + z * z
    a2 = (cx * cx + cy * cy) + cz * cz

    def bf(v):
        return v.astype(jnp.bfloat16).astype(jnp.float32)

    ab = (bf(cx) * bf(x) + bf(cy) * bf(y)) + bf(cz) * bf(z)
    d2 = jnp.maximum(a2 + b2 - 2.0 * ab, 0.0)
    dist_ref[...] = jnp.sqrt(d2)

    n3 = (jax.lax.broadcasted_iota(jnp.int32, (GB, NJ, NL), 1) * NL
          + jax.lax.broadcasted_iota(jnp.int32, (GB, NJ, NL), 2))
    laneio = jax.lax.broadcasted_iota(jnp.int32, (GB, NL), 1)
    kiota = jax.lax.broadcasted_iota(jnp.int32, (GB, GS), 1)
    inf = jnp.float32(jnp.inf)
    nbig = jnp.int32(n_total)

    def build(mk, nk):
        d = dist_ref[...]
        mkb = mk.reshape(GB, 1, 1)
        nkb = nk.reshape(GB, 1, 1)
        dx = jnp.where((d < mkb) | ((d == mkb) & (n3 <= nkb)), inf, d)
        m1 = jnp.min(dx, axis=1)
        a1 = jnp.min(jnp.where(dx == m1[:, None, :], n3, nbig), axis=1)
        dx = jnp.where(n3 == a1[:, None, :], inf, dx)
        m2 = jnp.min(dx, axis=1)
        a2_ = jnp.min(jnp.where(dx == m2[:, None, :], n3, nbig), axis=1)
        dx = jnp.where(n3 == a2_[:, None, :], inf, dx)
        m3 = jnp.min(dx, axis=1)
        a3 = jnp.min(jnp.where(dx == m3[:, None, :], n3, nbig), axis=1)
        c = ((m1 != inf).astype(jnp.int32) + (m2 != inf).astype(jnp.int32)
             + (m3 != inf).astype(jnp.int32))
        m1_ref[...] = m1
        a1_ref[...] = a1
        m2_ref[...] = m2
        a2_ref[...] = a2_
        m3_ref[...] = m3
        a3_ref[...] = a3
        c_ref[...] = c

    def body(k, aknn):
        m1 = m1_ref[...]
        a1 = a1_ref[...]
        m = jnp.min(m1, axis=1, keepdims=True)
        idx = jnp.min(jnp.where(m1 == m, a1, nbig), axis=1,
                      keepdims=True)
        aknn = jnp.where(kiota == k, idx + b * n_total, aknn)
        sel = laneio == jnp.bitwise_and(idx, jnp.int32(NL - 1))
        m2 = m2_ref[...]
        a2_ = a2_ref[...]
        m1_ref[...] = jnp.where(sel, m2, m1)
        a1_ref[...] = jnp.where(sel, a2_, a1)
        m2_ref[...] = jnp.where(sel, m3_ref[...], m2)
        a2_ref[...] = jnp.where(sel, a3_ref[...], a2_)
        m3_ref[...] = jnp.where(sel, inf, m3_ref[...])
        a3_ref[...] = jnp.where(sel, nbig, a3_ref[...])
        cn = c_ref[...] - sel.astype(jnp.int32)
        c_ref[...] = cn

        @pl.when(jnp.min(cn) == 0)
        def _():
            build(m, idx)

        return aknn

    build(jnp.full((GB, 1), -jnp.inf, jnp.float32),
          jnp.full((GB, 1), -1, jnp.int32))
    zi = jnp.zeros((GB, GS), jnp.int32)
    knn_ref[0] = jax.lax.fori_loop(0, GS, body, zi)


def _make_sc_gather(total, pad):
    info = plsc.get_sparse_core_info()
    nw = info.num_cores * info.num_subcores
    rpw = total // nw

    @functools.partial(
        pl.kernel,
        mesh=plsc.VectorSubcoreMesh(core_axis_name="c", subcore_axis_name="s"),
        compiler_params=pltpu.CompilerParams(use_tc_tiling_on_sc=False),
        out_type=jax.ShapeDtypeStruct((total, pad), jnp.float32),
        scratch_types=[
            pltpu.VMEM((rpw,), jnp.int32),
            pltpu.VMEM((rpw, pad), jnp.float32),
            pltpu.SemaphoreType.DMA,
        ],
    )
    def gather_k(idx_hbm, tab_hbm, out_hbm, idx_v, rows_v, sem):
        wid = lax.axis_index("s") * info.num_cores + lax.axis_index("c")
        base = wid * rpw
        pltpu.sync_copy(idx_hbm.at[pl.ds(base, rpw)], idx_v)
        pltpu.async_copy(tab_hbm.at[idx_v], rows_v, sem).wait()
        pltpu.sync_copy(rows_v, out_hbm.at[pl.ds(base, rpw)])

    return gather_k


@jax.jit
def kernel(xyz):
    B, N, _ = xyz.shape
    x = xyz[:, :, 0]
    y = xyz[:, :, 1]
    z = xyz[:, :, 2]
    xr = x.reshape(B, FR, N // FR)
    yr = y.reshape(B, FR, N // FR)
    zr = z.reshape(B, FR, N // FR)

    cplanes = pl.pallas_call(
        _fps_body,
        grid=(B,),
        in_specs=[pl.BlockSpec((1, FR, N // FR), lambda b: (b, 0, 0))] * 3,
        out_specs=[pl.BlockSpec((1, 1, NG), lambda b: (b, 0, 0))] * 3,
        out_shape=[jax.ShapeDtypeStruct((B, 1, NG), jnp.float32)] * 3,
        scratch_shapes=[pltpu.VMEM((FR, N // FR), jnp.float32)],
    )(xr, yr, zr)
    cx, cy, cz = (c.reshape(B, NG) for c in cplanes)
    cxr = cx.reshape(B, NG // GB, GB, 1)
    cyr = cy.reshape(B, NG // GB, GB, 1)
    czr = cz.reshape(B, NG // GB, GB, 1)

    knn = pl.pallas_call(
        _knn_body,
        grid=(B, NG // GB),
        in_specs=(
            [pl.BlockSpec((1, NJ, NL), lambda b, g: (b, 0, 0))] * 3
            + [pl.BlockSpec((1, 1, GB, 1), lambda b, g: (b, g, 0, 0))] * 3
        ),
        out_specs=pl.BlockSpec((1, GB, GS), lambda b, g: (b, g, 0)),
        out_shape=jax.ShapeDtypeStruct((B, NG, GS), jnp.int32),
        scratch_shapes=[pltpu.VMEM((GB, NJ, NL), jnp.float32),
                        pltpu.VMEM((GB, NL), jnp.float32),
                        pltpu.VMEM((GB, NL), jnp.int32),
                        pltpu.VMEM((GB, NL), jnp.float32),
                        pltpu.VMEM((GB, NL), jnp.int32),
                        pltpu.VMEM((GB, NL), jnp.float32),
                        pltpu.VMEM((GB, NL), jnp.int32),
                        pltpu.VMEM((GB, NL), jnp.int32)],
    )(x.reshape(B, NJ, NL), y.reshape(B, NJ, NL), z.reshape(B, NJ, NL),
      cxr, cyr, czr)

    knn_idx_flat = knn.reshape(-1)
    total = B * NG * GS
    pad = 8
    table = jnp.concatenate(
        [xyz.reshape(B * N, 3),
         jnp.zeros((B * N, pad - 3), jnp.float32)], axis=1)
    rows = _make_sc_gather(total, pad)(knn_idx_flat, table)
    nbr_xyz = rows[:, :3].reshape(B, NG, GS, 3)
    return knn_idx_flat, nbr_xyz
